# Initial kernel scaffold; baseline (speedup 1.0000x reference)
import jax
import jax.numpy as jnp
from jax.experimental import pallas as pl


def kernel(inputs, edge_index, edge_weight, W, b):
    raise NotImplementedError("write your pallas kernel here")


# speedup vs baseline: 4.2470x; 4.2470x over previous
import functools

import jax
import jax.numpy as jnp
from jax import lax
from jax.experimental import pallas as pl
from jax.experimental.pallas import tpu as pltpu
from jax.experimental.pallas import tpu_sc as plsc

N_NODES = 10000
D_FEAT = 128
UNITS = 128

LANES = 16
CHUNK = 128
N_WORKERS = 32
ROWS_PER_TILE = 624
TAIL_ROWS = N_NODES - 16 * ROWS_PER_TILE


def _mm_body(x_ref, w_ref, o_ref):
    o_ref[...] = jnp.dot(x_ref[...], w_ref[...],
                         preferred_element_type=jnp.float32)


def _matmul(x, w):
    m = x.shape[0]
    blk = 1000
    grid = m // blk
    return pl.pallas_call(
        _mm_body,
        grid=(grid,),
        in_specs=[
            pl.BlockSpec((blk, D_FEAT), lambda i: (i, 0)),
            pl.BlockSpec((D_FEAT, UNITS), lambda i: (0, 0)),
        ],
        out_specs=pl.BlockSpec((blk, UNITS), lambda i: (i, 0)),
        out_shape=jax.ShapeDtypeStruct((m, UNITS), jnp.float32),
    )(x, w)


def _combine_body(p0_ref, p1_ref, b_ref, o_ref):
    o_ref[...] = p0_ref[...] + p1_ref[...] + b_ref[...]


def _combine(p0, p1, b2d):
    m = p0.shape[0]
    blk = 1000
    grid = m // blk
    return pl.pallas_call(
        _combine_body,
        grid=(grid,),
        in_specs=[
            pl.BlockSpec((blk, UNITS), lambda i: (i, 0)),
            pl.BlockSpec((blk, UNITS), lambda i: (i, 0)),
            pl.BlockSpec((1, UNITS), lambda i: (0, 0)),
        ],
        out_specs=pl.BlockSpec((blk, UNITS), lambda i: (i, 0)),
        out_shape=jax.ShapeDtypeStruct((m, UNITS), jnp.float32),
    )(p0, p1, b2d)


def _make_sc_kernel(n_chunks):
    mesh = plsc.VectorSubcoreMesh(core_axis_name="c", subcore_axis_name="s")

    @functools.partial(
        pl.kernel,
        mesh=mesh,
        out_type=jax.ShapeDtypeStruct((2, N_NODES, UNITS), jnp.float32),
        scratch_types=[
            pltpu.VMEM((n_chunks, CHUNK), jnp.int32),
            pltpu.VMEM((n_chunks, CHUNK), jnp.int32),
            pltpu.VMEM((n_chunks, CHUNK), jnp.float32),
            pltpu.VMEM((CHUNK, UNITS), jnp.float32),
            pltpu.VMEM_SHARED((N_NODES, UNITS), jnp.float32),
            pltpu.SemaphoreType.DMA,
        ],
    )
    def sc_kernel(support_hbm, src_hbm, dst_hbm, w_hbm, zeros_hbm, out_hbm,
                  src_v, dst_v, w_v, buf, acc, sem):
        cid = lax.axis_index("c")
        sid = lax.axis_index("s")
        wid = cid * 16 + sid

        row0 = sid * ROWS_PER_TILE
        pltpu.sync_copy(zeros_hbm.at[pl.ds(row0, ROWS_PER_TILE)],
                        acc.at[pl.ds(row0, ROWS_PER_TILE)])

        @pl.when(sid == 0)
        def _():
            tail0 = 16 * ROWS_PER_TILE
            pltpu.sync_copy(zeros_hbm.at[pl.ds(tail0, TAIL_ROWS)],
                            acc.at[pl.ds(tail0, TAIL_ROWS)])

        pltpu.sync_copy(src_hbm.at[wid], src_v)
        pltpu.sync_copy(dst_hbm.at[wid], dst_v)
        pltpu.sync_copy(w_hbm.at[wid], w_v)

        plsc.subcore_barrier()

        def chunk_body(j, carry):
            pltpu.async_copy(support_hbm.at[src_v.at[j]], buf, sem).wait()

            def group_body(g, c2):
                wg = w_v[j, pl.ds(g * LANES, LANES)]
                for l in range(LANES):
                    e = g * LANES + l
                    wvec = jnp.full((LANES,), wg[l], dtype=jnp.float32)
                    for c in range(UNITS // LANES):
                        sl = pl.ds(c * LANES, LANES)
                        buf[e, sl] = buf[e, sl] * wvec
                return c2

            lax.fori_loop(0, CHUNK // LANES, group_body, 0)

            pltpu.sync_copy(buf, acc.at[dst_v.at[j]], add=True)
            return carry

        lax.fori_loop(0, n_chunks, chunk_body, 0)

        plsc.subcore_barrier()

        pltpu.sync_copy(acc.at[pl.ds(row0, ROWS_PER_TILE)],
                        out_hbm.at[cid, pl.ds(row0, ROWS_PER_TILE)])

        @pl.when(sid == 0)
        def _():
            tail0 = 16 * ROWS_PER_TILE
            pltpu.sync_copy(acc.at[pl.ds(tail0, TAIL_ROWS)],
                            out_hbm.at[cid, pl.ds(tail0, TAIL_ROWS)])

    return sc_kernel


@jax.jit
def kernel(inputs, edge_index, edge_weight, W, b):
    n_edges = edge_index.shape[1]
    n_chunks = -(-n_edges // (N_WORKERS * CHUNK))
    padded = N_WORKERS * n_chunks * CHUNK
    pad = padded - n_edges

    src = jnp.pad(edge_index[0], (0, pad)).reshape(N_WORKERS, n_chunks, CHUNK)
    dst = jnp.pad(edge_index[1], (0, pad)).reshape(N_WORKERS, n_chunks, CHUNK)
    w = jnp.pad(edge_weight, (0, pad)).reshape(N_WORKERS, n_chunks, CHUNK)

    support = _matmul(inputs, W)
    zeros = jnp.zeros((N_NODES, UNITS), jnp.float32)
    parts = _make_sc_kernel(n_chunks)(support, src, dst, w, zeros)
    return _combine(parts[0], parts[1], b.reshape(1, UNITS))
